# baseline (speedup 1.0000x reference)
import functools

import jax
import jax.numpy as jnp
from jax import lax
from jax.experimental import pallas as pl
from jax.experimental.pallas import tpu as pltpu
from jax.experimental.pallas import tpu_sc as plsc

N = 10000
D = 128
E = 320000
NC = 2
NS = 16
NW = NC * NS
CPW = D // NW
CHE = 2048
ECH = 160
EP = ECH * CHE
GU = 16
NP = 10240
BN_EPS = 1e-5
BLK = 1024
GRID = NP // BLK

_sc_mesh = plsc.VectorSubcoreMesh(
    core_axis_name="c", subcore_axis_name="s", num_cores=NC, num_subcores=NS)


@functools.partial(
    pl.kernel,
    out_type=jax.ShapeDtypeStruct((D, NP), jnp.float32),
    mesh=_sc_mesh,
    compiler_params=pltpu.CompilerParams(needs_layout_passes=False),
    scratch_types=[
        pltpu.VMEM((CPW * NP,), jnp.float32),
        pltpu.VMEM((CPW * NP,), jnp.float32),
        pltpu.VMEM((CHE,), jnp.int32),
        pltpu.VMEM((CHE,), jnp.int32),
        pltpu.VMEM((CHE,), jnp.int32),
        pltpu.VMEM((CHE,), jnp.int32),
        [pltpu.SemaphoreType.DMA] * 4,
    ],
)
def _sc_segsum(rt_hbm, s_hbm, d_hbm, z_hbm, st_hbm,
               rt, at, sb0, sb1, db0, db1, sems):
    c = lax.axis_index("c")
    s = lax.axis_index("s")
    w = c * NS + s
    for cc in range(CPW):
        pltpu.sync_copy(rt_hbm.at[w * CPW + cc], rt.at[pl.ds(cc * NP, NP)])
        pltpu.sync_copy(z_hbm, at.at[pl.ds(cc * NP, NP)])
    pltpu.async_copy(s_hbm.at[pl.ds(0, CHE)], sb0, sems[0])
    pltpu.async_copy(d_hbm.at[pl.ds(0, CHE)], db0, sems[1])
    pltpu.async_copy(s_hbm.at[pl.ds(CHE, CHE)], sb1, sems[2])
    pltpu.async_copy(d_hbm.at[pl.ds(CHE, CHE)], db1, sems[3])

    def outer(g, carry):
        for p, sb, db, ssem, dsem in (
                (0, sb0, db0, sems[0], sems[1]),
                (1, sb1, db1, sems[2], sems[3])):
            ch = g * 2 + p
            pltpu.make_async_copy(s_hbm.at[pl.ds(0, CHE)], sb, ssem).wait()
            pltpu.make_async_copy(d_hbm.at[pl.ds(0, CHE)], db, dsem).wait()

            @plsc.parallel_loop(0, CHE // 16, 1, unroll=GU)
            def _(t):
                off = t * 16
                srcs = sb[pl.ds(off, 16)]
                dsts = db[pl.ds(off, 16)]
                for cc in range(CPW):
                    v = plsc.load_gather(rt, [srcs + cc * NP])
                    plsc.addupdate_scatter(at, [dsts + cc * NP], v)

            @pl.when(ch + 2 < ECH)
            def _():
                off = (ch + 2) * CHE
                pltpu.async_copy(s_hbm.at[pl.ds(off, CHE)], sb, ssem)
                pltpu.async_copy(d_hbm.at[pl.ds(off, CHE)], db, dsem)
        return carry

    lax.fori_loop(0, ECH // 2, outer, 0)
    for cc in range(CPW):
        pltpu.sync_copy(at.at[pl.ds(cc * NP, NP)], st_hbm.at[w * CPW + cc])


def _prep_body(x_ref, xt_ref, rt_ref):
    xt = x_ref[...].T
    xt_ref[...] = xt
    rt_ref[...] = jnp.maximum(xt, 0.0)


def _mlp_body(eps_ref, xt_ref, pt_ref, w1_ref, b1_ref, w2_ref, b2_ref,
              yt_ref, st_ref):
    i = pl.program_id(0)
    z = xt_ref[...] * eps_ref[0, 0] + pt_ref[...]
    h = jnp.maximum(
        jnp.dot(w1_ref[...], z, preferred_element_type=jnp.float32)
        + b1_ref[...], 0.0)
    y = (jnp.dot(w2_ref[...], h, preferred_element_type=jnp.float32)
         + b2_ref[...])
    yt_ref[...] = y

    @pl.when(i == 0)
    def _():
        st_ref[...] = jnp.zeros_like(st_ref)

    col = lax.broadcasted_iota(jnp.int32, (D, BLK), 1) + i * BLK
    ym = jnp.where(col < N, y, 0.0)
    st = jnp.concatenate(
        [jnp.sum(ym, axis=1)[:, None], jnp.sum(ym * ym, axis=1)[:, None],
         jnp.zeros((D, 6), jnp.float32)], axis=1)
    st_ref[...] += st


def _mlp_res_body(eps_ref, xt_ref, pt_ref, w1_ref, b1_ref, w2_ref, b2_ref,
                  o_ref):
    z = xt_ref[...] * eps_ref[0, 0] + pt_ref[...]
    h = jnp.maximum(
        jnp.dot(w1_ref[...], z, preferred_element_type=jnp.float32)
        + b1_ref[...], 0.0)
    y = (jnp.dot(w2_ref[...], h, preferred_element_type=jnp.float32)
         + b2_ref[...] + xt_ref[...])
    o_ref[...] = y.T


def _bn_body(yt_ref, st_ref, g_ref, b_ref, x0_ref, xo_ref, rt_ref):
    stats = st_ref[...]
    mean = stats[:, 0:1] * (1.0 / N)
    var = stats[:, 1:2] * (1.0 / N) - mean * mean
    scale = lax.rsqrt(var + BN_EPS) * g_ref[...]
    xn = (yt_ref[...] - mean) * scale + b_ref[...] + x0_ref[...]
    xo_ref[...] = xn
    rt_ref[...] = jnp.maximum(xn, 0.0)


_row_spec = pl.BlockSpec((BLK, D), lambda i: (i, 0))
_t_spec = pl.BlockSpec((D, BLK), lambda i: (0, i))
_full_spec = pl.BlockSpec((D, D), lambda i: (0, 0))
_cvec_spec = pl.BlockSpec((D, 1), lambda i: (0, 0))
_st_spec = pl.BlockSpec((D, 8), lambda i: (0, 0))
_smem_spec = pl.BlockSpec(memory_space=pltpu.SMEM)

_prep_call = pl.pallas_call(
    _prep_body,
    grid=(GRID,),
    in_specs=[_row_spec],
    out_specs=[_t_spec, _t_spec],
    out_shape=[jax.ShapeDtypeStruct((D, NP), jnp.float32),
               jax.ShapeDtypeStruct((D, NP), jnp.float32)],
)

_mlp_call = pl.pallas_call(
    _mlp_body,
    grid=(GRID,),
    in_specs=[_smem_spec, _t_spec, _t_spec, _full_spec, _cvec_spec,
              _full_spec, _cvec_spec],
    out_specs=[_t_spec, _st_spec],
    out_shape=[jax.ShapeDtypeStruct((D, NP), jnp.float32),
               jax.ShapeDtypeStruct((D, 8), jnp.float32)],
)

_mlp_res_call = pl.pallas_call(
    _mlp_res_body,
    grid=(GRID,),
    in_specs=[_smem_spec, _t_spec, _t_spec, _full_spec, _cvec_spec,
              _full_spec, _cvec_spec],
    out_specs=_row_spec,
    out_shape=jax.ShapeDtypeStruct((NP, D), jnp.float32),
)

_bn_call = pl.pallas_call(
    _bn_body,
    grid=(GRID,),
    in_specs=[_t_spec, _st_spec, _cvec_spec, _cvec_spec, _t_spec],
    out_specs=[_t_spec, _t_spec],
    out_shape=[jax.ShapeDtypeStruct((D, NP), jnp.float32),
               jax.ShapeDtypeStruct((D, NP), jnp.float32)],
)


def kernel(X, edge_index, params):
    src = edge_index[0].astype(jnp.int32)
    dst = edge_index[1].astype(jnp.int32)
    pad = EP - E
    src_p = jnp.concatenate([src, jnp.zeros((pad,), jnp.int32)])
    dst_p = jnp.concatenate([dst, jnp.full((pad,), N, jnp.int32)])
    zrows = jnp.zeros((NP,), jnp.float32)

    Xp = jnp.pad(X, ((0, NP - N), (0, 0)))
    XT, RT = _prep_call(Xp)
    out = None
    for li, p in enumerate(params):
        scale = (1.0 + p['eps']).reshape(1, 1)
        w1t = p['W1'].T
        w2t = p['W2'].T
        b1 = p['b1'].reshape(D, 1)
        b2 = p['b2'].reshape(D, 1)
        ST = _sc_segsum(RT, src_p, dst_p, zrows)
        if li < len(params) - 1:
            YT, stats = _mlp_call(scale, XT, ST, w1t, b1, w2t, b2)
            XT, RT = _bn_call(YT, stats, p['gamma'].reshape(D, 1),
                              p['beta'].reshape(D, 1), XT)
        else:
            out = _mlp_res_call(scale, XT, ST, w1t, b1, w2t, b2)
    return out[:N]


# speedup vs baseline: 1.0866x; 1.0866x over previous
import functools

import jax
import jax.numpy as jnp
from jax import lax
from jax.experimental import pallas as pl
from jax.experimental.pallas import tpu as pltpu
from jax.experimental.pallas import tpu_sc as plsc

N = 10000
D = 128
E = 320000
NC = 2
NS = 16
NW = NC * NS
CPW = D // NW
CHE = 2048
ECH = 160
EP = ECH * CHE
GU = 8
NP = 10240
BN_EPS = 1e-5
BLK = 1024
GRID = NP // BLK

_sc_mesh = plsc.VectorSubcoreMesh(
    core_axis_name="c", subcore_axis_name="s", num_cores=NC, num_subcores=NS)


@functools.partial(
    pl.kernel,
    out_type=jax.ShapeDtypeStruct((D, NP), jnp.float32),
    mesh=_sc_mesh,
    compiler_params=pltpu.CompilerParams(needs_layout_passes=False),
    scratch_types=[
        pltpu.VMEM((CPW * NP,), jnp.float32),
        pltpu.VMEM((CPW * NP,), jnp.float32),
        pltpu.VMEM((CHE,), jnp.int32),
        pltpu.VMEM((CHE,), jnp.int32),
        pltpu.VMEM((CHE,), jnp.int32),
        pltpu.VMEM((CHE,), jnp.int32),
        [pltpu.SemaphoreType.DMA] * 4,
    ],
)
def _sc_segsum(rt_hbm, s_hbm, d_hbm, z_hbm, st_hbm,
               rt, at, sb0, sb1, db0, db1, sems):
    c = lax.axis_index("c")
    s = lax.axis_index("s")
    w = c * NS + s
    for cc in range(CPW):
        pltpu.sync_copy(rt_hbm.at[w * CPW + cc], rt.at[pl.ds(cc * NP, NP)])
        pltpu.sync_copy(z_hbm, at.at[pl.ds(cc * NP, NP)])
    pltpu.async_copy(s_hbm.at[pl.ds(0, CHE)], sb0, sems[0])
    pltpu.async_copy(d_hbm.at[pl.ds(0, CHE)], db0, sems[1])
    pltpu.async_copy(s_hbm.at[pl.ds(CHE, CHE)], sb1, sems[2])
    pltpu.async_copy(d_hbm.at[pl.ds(CHE, CHE)], db1, sems[3])

    def outer(g, carry):
        for p, sb, db, ssem, dsem in (
                (0, sb0, db0, sems[0], sems[1]),
                (1, sb1, db1, sems[2], sems[3])):
            ch = g * 2 + p
            pltpu.make_async_copy(s_hbm.at[pl.ds(0, CHE)], sb, ssem).wait()
            pltpu.make_async_copy(d_hbm.at[pl.ds(0, CHE)], db, dsem).wait()

            @plsc.parallel_loop(0, CHE // 16, 1, unroll=GU)
            def _(t):
                off = t * 16
                srcs = sb[pl.ds(off, 16)]
                dsts = db[pl.ds(off, 16)]
                for cc in range(CPW):
                    v = plsc.load_gather(rt, [srcs + cc * NP])
                    plsc.addupdate_scatter(at, [dsts + cc * NP], v)

            @pl.when(ch + 2 < ECH)
            def _():
                off = (ch + 2) * CHE
                pltpu.async_copy(s_hbm.at[pl.ds(off, CHE)], sb, ssem)
                pltpu.async_copy(d_hbm.at[pl.ds(off, CHE)], db, dsem)
        return carry

    lax.fori_loop(0, ECH // 2, outer, 0)
    for cc in range(CPW):
        pltpu.sync_copy(at.at[pl.ds(cc * NP, NP)], st_hbm.at[w * CPW + cc])


def _prep_body(x_ref, xt_ref, rt_ref):
    xt = x_ref[...].T
    xt_ref[...] = xt
    rt_ref[...] = jnp.maximum(xt, 0.0)


def _mlp_body(eps_ref, xt_ref, pt_ref, w1_ref, b1_ref, w2_ref, b2_ref,
              yt_ref, st_ref):
    i = pl.program_id(0)
    z = xt_ref[...] * eps_ref[0, 0] + pt_ref[...]
    h = jnp.maximum(
        jnp.dot(w1_ref[...], z, preferred_element_type=jnp.float32)
        + b1_ref[...], 0.0)
    y = (jnp.dot(w2_ref[...], h, preferred_element_type=jnp.float32)
         + b2_ref[...])
    yt_ref[...] = y

    @pl.when(i == 0)
    def _():
        st_ref[...] = jnp.zeros_like(st_ref)

    col = lax.broadcasted_iota(jnp.int32, (D, BLK), 1) + i * BLK
    ym = jnp.where(col < N, y, 0.0)
    st = jnp.concatenate(
        [jnp.sum(ym, axis=1)[:, None], jnp.sum(ym * ym, axis=1)[:, None],
         jnp.zeros((D, 6), jnp.float32)], axis=1)
    st_ref[...] += st


def _mlp_res_body(eps_ref, xt_ref, pt_ref, w1_ref, b1_ref, w2_ref, b2_ref,
                  o_ref):
    z = xt_ref[...] * eps_ref[0, 0] + pt_ref[...]
    h = jnp.maximum(
        jnp.dot(w1_ref[...], z, preferred_element_type=jnp.float32)
        + b1_ref[...], 0.0)
    y = (jnp.dot(w2_ref[...], h, preferred_element_type=jnp.float32)
         + b2_ref[...] + xt_ref[...])
    o_ref[...] = y.T


def _bn_body(yt_ref, st_ref, g_ref, b_ref, x0_ref, xo_ref, rt_ref):
    stats = st_ref[...]
    mean = stats[:, 0:1] * (1.0 / N)
    var = stats[:, 1:2] * (1.0 / N) - mean * mean
    scale = lax.rsqrt(var + BN_EPS) * g_ref[...]
    xn = (yt_ref[...] - mean) * scale + b_ref[...] + x0_ref[...]
    xo_ref[...] = xn
    rt_ref[...] = jnp.maximum(xn, 0.0)


_row_spec = pl.BlockSpec((BLK, D), lambda i: (i, 0))
_t_spec = pl.BlockSpec((D, BLK), lambda i: (0, i))
_full_spec = pl.BlockSpec((D, D), lambda i: (0, 0))
_cvec_spec = pl.BlockSpec((D, 1), lambda i: (0, 0))
_st_spec = pl.BlockSpec((D, 8), lambda i: (0, 0))
_smem_spec = pl.BlockSpec(memory_space=pltpu.SMEM)

_prep_call = pl.pallas_call(
    _prep_body,
    grid=(GRID,),
    in_specs=[_row_spec],
    out_specs=[_t_spec, _t_spec],
    out_shape=[jax.ShapeDtypeStruct((D, NP), jnp.float32),
               jax.ShapeDtypeStruct((D, NP), jnp.float32)],
)

_mlp_call = pl.pallas_call(
    _mlp_body,
    grid=(GRID,),
    in_specs=[_smem_spec, _t_spec, _t_spec, _full_spec, _cvec_spec,
              _full_spec, _cvec_spec],
    out_specs=[_t_spec, _st_spec],
    out_shape=[jax.ShapeDtypeStruct((D, NP), jnp.float32),
               jax.ShapeDtypeStruct((D, 8), jnp.float32)],
)

_mlp_res_call = pl.pallas_call(
    _mlp_res_body,
    grid=(GRID,),
    in_specs=[_smem_spec, _t_spec, _t_spec, _full_spec, _cvec_spec,
              _full_spec, _cvec_spec],
    out_specs=_row_spec,
    out_shape=jax.ShapeDtypeStruct((NP, D), jnp.float32),
)

_bn_call = pl.pallas_call(
    _bn_body,
    grid=(GRID,),
    in_specs=[_t_spec, _st_spec, _cvec_spec, _cvec_spec, _t_spec],
    out_specs=[_t_spec, _t_spec],
    out_shape=[jax.ShapeDtypeStruct((D, NP), jnp.float32),
               jax.ShapeDtypeStruct((D, NP), jnp.float32)],
)


def kernel(X, edge_index, params):
    src = edge_index[0].astype(jnp.int32)
    dst = edge_index[1].astype(jnp.int32)
    pad = EP - E
    src_p = jnp.concatenate([src, jnp.zeros((pad,), jnp.int32)])
    dst_p = jnp.concatenate([dst, jnp.full((pad,), N, jnp.int32)])
    zrows = jnp.zeros((NP,), jnp.float32)

    Xp = jnp.pad(X, ((0, NP - N), (0, 0)))
    XT, RT = _prep_call(Xp)
    out = None
    for li, p in enumerate(params):
        scale = (1.0 + p['eps']).reshape(1, 1)
        w1t = p['W1'].T
        w2t = p['W2'].T
        b1 = p['b1'].reshape(D, 1)
        b2 = p['b2'].reshape(D, 1)
        ST = _sc_segsum(RT, src_p, dst_p, zrows)
        if li < len(params) - 1:
            YT, stats = _mlp_call(scale, XT, ST, w1t, b1, w2t, b2)
            XT, RT = _bn_call(YT, stats, p['gamma'].reshape(D, 1),
                              p['beta'].reshape(D, 1), XT)
        else:
            out = _mlp_res_call(scale, XT, ST, w1t, b1, w2t, b2)
    return out[:N]


# speedup vs baseline: 1.0879x; 1.0012x over previous
import functools

import jax
import jax.numpy as jnp
from jax import lax
from jax.experimental import pallas as pl
from jax.experimental.pallas import tpu as pltpu
from jax.experimental.pallas import tpu_sc as plsc

N = 10000
D = 128
E = 320000
NC = 2
NS = 16
NW = NC * NS
CPW = D // NW
CHE = 4096
ECH = 80
EP = ECH * CHE
GU = 8
NP = 10240
BN_EPS = 1e-5
BLK = 1024
GRID = NP // BLK

_sc_mesh = plsc.VectorSubcoreMesh(
    core_axis_name="c", subcore_axis_name="s", num_cores=NC, num_subcores=NS)


@functools.partial(
    pl.kernel,
    out_type=jax.ShapeDtypeStruct((D, NP), jnp.float32),
    mesh=_sc_mesh,
    compiler_params=pltpu.CompilerParams(needs_layout_passes=False),
    scratch_types=[
        pltpu.VMEM((CPW * NP,), jnp.float32),
        pltpu.VMEM((CPW * NP,), jnp.float32),
        pltpu.VMEM((CHE,), jnp.int32),
        pltpu.VMEM((CHE,), jnp.int32),
        pltpu.VMEM((CHE,), jnp.int32),
        pltpu.VMEM((CHE,), jnp.int32),
        [pltpu.SemaphoreType.DMA] * 4,
    ],
)
def _sc_segsum(rt_hbm, s_hbm, d_hbm, z_hbm, st_hbm,
               rt, at, sb0, sb1, db0, db1, sems):
    c = lax.axis_index("c")
    s = lax.axis_index("s")
    w = c * NS + s
    for cc in range(CPW):
        pltpu.sync_copy(rt_hbm.at[w * CPW + cc], rt.at[pl.ds(cc * NP, NP)])
        pltpu.sync_copy(z_hbm, at.at[pl.ds(cc * NP, NP)])
    pltpu.async_copy(s_hbm.at[pl.ds(0, CHE)], sb0, sems[0])
    pltpu.async_copy(d_hbm.at[pl.ds(0, CHE)], db0, sems[1])
    pltpu.async_copy(s_hbm.at[pl.ds(CHE, CHE)], sb1, sems[2])
    pltpu.async_copy(d_hbm.at[pl.ds(CHE, CHE)], db1, sems[3])

    def outer(g, carry):
        for p, sb, db, ssem, dsem in (
                (0, sb0, db0, sems[0], sems[1]),
                (1, sb1, db1, sems[2], sems[3])):
            ch = g * 2 + p
            pltpu.make_async_copy(s_hbm.at[pl.ds(0, CHE)], sb, ssem).wait()
            pltpu.make_async_copy(d_hbm.at[pl.ds(0, CHE)], db, dsem).wait()

            @plsc.parallel_loop(0, CHE // 16, 1, unroll=GU)
            def _(t):
                off = t * 16
                srcs = sb[pl.ds(off, 16)]
                dsts = db[pl.ds(off, 16)]
                for cc in range(CPW):
                    v = plsc.load_gather(rt, [srcs + cc * NP])
                    plsc.addupdate_scatter(at, [dsts + cc * NP], v)

            @pl.when(ch + 2 < ECH)
            def _():
                off = (ch + 2) * CHE
                pltpu.async_copy(s_hbm.at[pl.ds(off, CHE)], sb, ssem)
                pltpu.async_copy(d_hbm.at[pl.ds(off, CHE)], db, dsem)
        return carry

    lax.fori_loop(0, ECH // 2, outer, 0)
    for cc in range(CPW):
        pltpu.sync_copy(at.at[pl.ds(cc * NP, NP)], st_hbm.at[w * CPW + cc])


def _prep_body(x_ref, xt_ref, rt_ref):
    xt = x_ref[...].T
    xt_ref[...] = xt
    rt_ref[...] = jnp.maximum(xt, 0.0)


def _mlp_body(eps_ref, xt_ref, pt_ref, w1_ref, b1_ref, w2_ref, b2_ref,
              yt_ref, st_ref):
    i = pl.program_id(0)
    z = xt_ref[...] * eps_ref[0, 0] + pt_ref[...]
    h = jnp.maximum(
        jnp.dot(w1_ref[...], z, preferred_element_type=jnp.float32)
        + b1_ref[...], 0.0)
    y = (jnp.dot(w2_ref[...], h, preferred_element_type=jnp.float32)
         + b2_ref[...])
    yt_ref[...] = y

    @pl.when(i == 0)
    def _():
        st_ref[...] = jnp.zeros_like(st_ref)

    col = lax.broadcasted_iota(jnp.int32, (D, BLK), 1) + i * BLK
    ym = jnp.where(col < N, y, 0.0)
    st = jnp.concatenate(
        [jnp.sum(ym, axis=1)[:, None], jnp.sum(ym * ym, axis=1)[:, None],
         jnp.zeros((D, 6), jnp.float32)], axis=1)
    st_ref[...] += st


def _mlp_res_body(eps_ref, xt_ref, pt_ref, w1_ref, b1_ref, w2_ref, b2_ref,
                  o_ref):
    z = xt_ref[...] * eps_ref[0, 0] + pt_ref[...]
    h = jnp.maximum(
        jnp.dot(w1_ref[...], z, preferred_element_type=jnp.float32)
        + b1_ref[...], 0.0)
    y = (jnp.dot(w2_ref[...], h, preferred_element_type=jnp.float32)
         + b2_ref[...] + xt_ref[...])
    o_ref[...] = y.T


def _bn_body(yt_ref, st_ref, g_ref, b_ref, x0_ref, xo_ref, rt_ref):
    stats = st_ref[...]
    mean = stats[:, 0:1] * (1.0 / N)
    var = stats[:, 1:2] * (1.0 / N) - mean * mean
    scale = lax.rsqrt(var + BN_EPS) * g_ref[...]
    xn = (yt_ref[...] - mean) * scale + b_ref[...] + x0_ref[...]
    xo_ref[...] = xn
    rt_ref[...] = jnp.maximum(xn, 0.0)


_row_spec = pl.BlockSpec((BLK, D), lambda i: (i, 0))
_t_spec = pl.BlockSpec((D, BLK), lambda i: (0, i))
_full_spec = pl.BlockSpec((D, D), lambda i: (0, 0))
_cvec_spec = pl.BlockSpec((D, 1), lambda i: (0, 0))
_st_spec = pl.BlockSpec((D, 8), lambda i: (0, 0))
_smem_spec = pl.BlockSpec(memory_space=pltpu.SMEM)

_prep_call = pl.pallas_call(
    _prep_body,
    grid=(GRID,),
    in_specs=[_row_spec],
    out_specs=[_t_spec, _t_spec],
    out_shape=[jax.ShapeDtypeStruct((D, NP), jnp.float32),
               jax.ShapeDtypeStruct((D, NP), jnp.float32)],
)

_mlp_call = pl.pallas_call(
    _mlp_body,
    grid=(GRID,),
    in_specs=[_smem_spec, _t_spec, _t_spec, _full_spec, _cvec_spec,
              _full_spec, _cvec_spec],
    out_specs=[_t_spec, _st_spec],
    out_shape=[jax.ShapeDtypeStruct((D, NP), jnp.float32),
               jax.ShapeDtypeStruct((D, 8), jnp.float32)],
)

_mlp_res_call = pl.pallas_call(
    _mlp_res_body,
    grid=(GRID,),
    in_specs=[_smem_spec, _t_spec, _t_spec, _full_spec, _cvec_spec,
              _full_spec, _cvec_spec],
    out_specs=_row_spec,
    out_shape=jax.ShapeDtypeStruct((NP, D), jnp.float32),
)

_bn_call = pl.pallas_call(
    _bn_body,
    grid=(GRID,),
    in_specs=[_t_spec, _st_spec, _cvec_spec, _cvec_spec, _t_spec],
    out_specs=[_t_spec, _t_spec],
    out_shape=[jax.ShapeDtypeStruct((D, NP), jnp.float32),
               jax.ShapeDtypeStruct((D, NP), jnp.float32)],
)


def kernel(X, edge_index, params):
    src = edge_index[0].astype(jnp.int32)
    dst = edge_index[1].astype(jnp.int32)
    pad = EP - E
    src_p = jnp.concatenate([src, jnp.zeros((pad,), jnp.int32)])
    dst_p = jnp.concatenate([dst, jnp.full((pad,), N, jnp.int32)])
    zrows = jnp.zeros((NP,), jnp.float32)

    Xp = jnp.pad(X, ((0, NP - N), (0, 0)))
    XT, RT = _prep_call(Xp)
    out = None
    for li, p in enumerate(params):
        scale = (1.0 + p['eps']).reshape(1, 1)
        w1t = p['W1'].T
        w2t = p['W2'].T
        b1 = p['b1'].reshape(D, 1)
        b2 = p['b2'].reshape(D, 1)
        ST = _sc_segsum(RT, src_p, dst_p, zrows)
        if li < len(params) - 1:
            YT, stats = _mlp_call(scale, XT, ST, w1t, b1, w2t, b2)
            XT, RT = _bn_call(YT, stats, p['gamma'].reshape(D, 1),
                              p['beta'].reshape(D, 1), XT)
        else:
            out = _mlp_res_call(scale, XT, ST, w1t, b1, w2t, b2)
    return out[:N]
